# Initial kernel scaffold; baseline (speedup 1.0000x reference)
import jax
import jax.numpy as jnp
from jax.experimental import pallas as pl


def kernel(pos_start, pos_end, emb_weight):
    raise NotImplementedError("write your pallas kernel here")


# speedup vs baseline: 1.4724x; 1.4724x over previous
import functools

import jax
import jax.numpy as jnp
from jax import lax
from jax.experimental import pallas as pl
from jax.experimental.pallas import tpu as pltpu
from jax.experimental.pallas import tpu_sc as plsc

N_TIME = 8192
BINS = 10000
OUT_WIDTH = 2048
BATCH = 4

_TOTAL_ROWS = BATCH * N_TIME


def _build_sc_call():
    info = plsc.get_sparse_core_info()
    nc, ns, nl = info.num_cores, info.num_subcores, info.num_lanes
    nw = nc * ns
    rows_per_w = _TOTAL_ROWS // nw
    chunk = 16
    n_chunks = rows_per_w // chunk

    mesh = plsc.VectorSubcoreMesh(core_axis_name="c", subcore_axis_name="s")

    @functools.partial(
        pl.kernel,
        mesh=mesh,
        out_type=jax.ShapeDtypeStruct((_TOTAL_ROWS, OUT_WIDTH), jnp.float32),
        scratch_types=[
            pltpu.VMEM((2, 16), jnp.float32),
            pltpu.VMEM((chunk, OUT_WIDTH), jnp.float32),
            pltpu.SemaphoreType.DMA,
        ],
    )
    def sc_kernel(params_hbm, table_hbm, out_hbm, params_v, buf_v, sem):
        wid = lax.axis_index("s") * nc + lax.axis_index("c")
        base0 = wid * rows_per_w
        t0 = base0 % N_TIME

        pltpu.sync_copy(params_hbm.at[wid], params_v)
        sv = params_v[0, :]
        dv = params_v[1, :]

        lane = lax.iota(jnp.int32, nl).astype(jnp.float32)

        def body(j, _):
            t_base = (t0 + j * chunk).astype(jnp.float32)
            tv = t_base + lane
            pos = sv + dv * (tv * (1.0 / N_TIME))
            idxv = (pos * float(BINS)).astype(jnp.int32)
            pltpu.async_copy(table_hbm.at[idxv], buf_v, sem).wait()
            pltpu.sync_copy(buf_v, out_hbm.at[pl.ds(base0 + j * chunk, chunk)])
            return 0

        lax.fori_loop(0, n_chunks, body, 0)

    return sc_kernel


def kernel(pos_start, pos_end, emb_weight):
    s = pos_start.reshape(BATCH)
    d = pos_end.reshape(BATCH) - s
    s_rep = jnp.repeat(s, 8)
    d_rep = jnp.repeat(d, 8)
    params = jnp.stack([s_rep, d_rep], axis=1)
    params = jnp.broadcast_to(params[:, :, None], (32, 2, 16))
    sc_call = _build_sc_call()
    out = sc_call(params, emb_weight)
    return out.reshape(BATCH, N_TIME, OUT_WIDTH)


# speedup vs baseline: 1.7704x; 1.2024x over previous
import functools

import jax
import jax.numpy as jnp
from jax import lax
from jax.experimental import pallas as pl
from jax.experimental.pallas import tpu as pltpu
from jax.experimental.pallas import tpu_sc as plsc

N_TIME = 8192
BINS = 10000
OUT_WIDTH = 2048
BATCH = 4

_TOTAL_ROWS = BATCH * N_TIME
_CHUNK = 16
_NBUF = 2


def _build_sc_call():
    info = plsc.get_sparse_core_info()
    nc, ns, nl = info.num_cores, info.num_subcores, info.num_lanes
    nw = nc * ns
    rows_per_w = _TOTAL_ROWS // nw
    n_chunks = rows_per_w // _CHUNK
    n_groups = n_chunks // _NBUF

    mesh = plsc.VectorSubcoreMesh(core_axis_name="c", subcore_axis_name="s")

    @functools.partial(
        pl.kernel,
        mesh=mesh,
        out_type=jax.ShapeDtypeStruct((_TOTAL_ROWS, OUT_WIDTH), jnp.float32),
        scratch_types=(
            [pltpu.VMEM((2, 16), jnp.float32)]
            + [pltpu.VMEM((_CHUNK, OUT_WIDTH), jnp.float32) for _ in range(_NBUF)]
            + [pltpu.SemaphoreType.DMA for _ in range(2 * _NBUF)]
        ),
    )
    def sc_kernel(params_hbm, table_hbm, out_hbm, params_v, *rest):
        bufs = rest[:_NBUF]
        gsem = rest[_NBUF:2 * _NBUF]
        wsem = rest[2 * _NBUF:]

        wid = lax.axis_index("s") * nc + lax.axis_index("c")
        base0 = wid * rows_per_w

        pltpu.sync_copy(params_hbm.at[wid], params_v)
        sv = params_v[0, :]
        dv = params_v[1, :]

        lane = lax.iota(jnp.int32, nl).astype(jnp.float32)

        def idx_of(j):
            t = (base0 % N_TIME + j * _CHUNK).astype(jnp.float32)
            tv = t + lane
            pos = sv + dv * (tv * (1.0 / N_TIME))
            idxv = (pos * float(BINS)).astype(jnp.int32)
            return idxv

        def gather_desc(j, b):
            return pltpu.make_async_copy(
                table_hbm.at[idx_of(j)], bufs[b], gsem[b]
            )

        for b in range(_NBUF):
            gather_desc(b, b).start()

        def body(g, _):
            writes = []
            for b in range(_NBUF):
                j = g * _NBUF + b
                gather_desc(j, b).wait()
                wd = pltpu.make_async_copy(
                    bufs[b],
                    out_hbm.at[pl.ds(base0 + j * _CHUNK, _CHUNK)],
                    wsem[b],
                )
                wd.start()
                writes.append(wd)
            for b in range(_NBUF):
                jn = (g + 1) * _NBUF + b
                writes[b].wait()

                @pl.when(jn < n_chunks)
                def _():
                    gather_desc(jn, b).start()

            return 0

        lax.fori_loop(0, n_groups, body, 0)

    return sc_kernel


def kernel(pos_start, pos_end, emb_weight):
    s = pos_start.reshape(BATCH)
    d = pos_end.reshape(BATCH) - s
    s_rep = jnp.repeat(s, 8)
    d_rep = jnp.repeat(d, 8)
    params = jnp.stack([s_rep, d_rep], axis=1)
    params = jnp.broadcast_to(params[:, :, None], (32, 2, 16))
    sc_call = _build_sc_call()
    out = sc_call(params, emb_weight)
    return out.reshape(BATCH, N_TIME, OUT_WIDTH)


# speedup vs baseline: 1.7716x; 1.0007x over previous
import functools

import jax
import jax.numpy as jnp
from jax import lax
from jax.experimental import pallas as pl
from jax.experimental.pallas import tpu as pltpu
from jax.experimental.pallas import tpu_sc as plsc

N_TIME = 8192
BINS = 10000
OUT_WIDTH = 2048
BATCH = 4

_TOTAL_ROWS = BATCH * N_TIME
_CHUNK = 16
_NBUF = 3


def _build_sc_call():
    info = plsc.get_sparse_core_info()
    nc, ns, nl = info.num_cores, info.num_subcores, info.num_lanes
    nw = nc * ns
    rows_per_w = _TOTAL_ROWS // nw
    n_chunks = rows_per_w // _CHUNK
    n_groups = (n_chunks - 1) // _NBUF
    n_ring = n_groups * _NBUF

    mesh = plsc.VectorSubcoreMesh(core_axis_name="c", subcore_axis_name="s")

    @functools.partial(
        pl.kernel,
        mesh=mesh,
        out_type=jax.ShapeDtypeStruct((_TOTAL_ROWS, OUT_WIDTH), jnp.float32),
        scratch_types=(
            [pltpu.VMEM((2, 16), jnp.float32)]
            + [pltpu.VMEM((_CHUNK, OUT_WIDTH), jnp.float32) for _ in range(_NBUF)]
            + [pltpu.SemaphoreType.DMA for _ in range(2 * _NBUF)]
        ),
    )
    def sc_kernel(params_hbm, table_hbm, out_hbm, params_v, *rest):
        bufs = rest[:_NBUF]
        gsem = rest[_NBUF:2 * _NBUF]
        wsem = rest[2 * _NBUF:]

        wid = lax.axis_index("s") * nc + lax.axis_index("c")
        base0 = wid * rows_per_w

        pltpu.sync_copy(params_hbm.at[wid], params_v)
        sv = params_v[0, :]
        dv = params_v[1, :]

        lane = lax.iota(jnp.int32, nl).astype(jnp.float32)

        def idx_of(j):
            t = (base0 % N_TIME + j * _CHUNK).astype(jnp.float32)
            tv = t + lane
            pos = sv + dv * (tv * (1.0 / N_TIME))
            idxv = (pos * float(BINS)).astype(jnp.int32)
            return idxv

        def gather_desc(j, b):
            return pltpu.make_async_copy(
                table_hbm.at[idx_of(j)], bufs[b], gsem[b]
            )

        for b in range(_NBUF):
            gather_desc(b, b).start()

        def body(g, _):
            writes = []
            for b in range(_NBUF):
                j = g * _NBUF + b
                gather_desc(j, b).wait()
                wd = pltpu.make_async_copy(
                    bufs[b],
                    out_hbm.at[pl.ds(base0 + j * _CHUNK, _CHUNK)],
                    wsem[b],
                )
                wd.start()
                writes.append(wd)
            for b in range(_NBUF):
                jn = (g + 1) * _NBUF + b
                writes[b].wait()

                @pl.when(jn < n_ring)
                def _():
                    gather_desc(jn, b).start()

            return 0

        lax.fori_loop(0, n_groups, body, 0)

        for j in range(n_ring, n_chunks):
            gd = gather_desc(j, 0)
            gd.start()
            gd.wait()
            wd = pltpu.make_async_copy(
                bufs[0],
                out_hbm.at[pl.ds(base0 + j * _CHUNK, _CHUNK)],
                wsem[0],
            )
            wd.start()
            wd.wait()

    return sc_kernel


def kernel(pos_start, pos_end, emb_weight):
    s = pos_start.reshape(BATCH)
    d = pos_end.reshape(BATCH) - s
    s_rep = jnp.repeat(s, 8)
    d_rep = jnp.repeat(d, 8)
    params = jnp.stack([s_rep, d_rep], axis=1)
    params = jnp.broadcast_to(params[:, :, None], (32, 2, 16))
    sc_call = _build_sc_call()
    out = sc_call(params, emb_weight)
    return out.reshape(BATCH, N_TIME, OUT_WIDTH)


# speedup vs baseline: 1.8127x; 1.0232x over previous
import functools

import jax
import jax.numpy as jnp
from jax import lax
from jax.experimental import pallas as pl
from jax.experimental.pallas import tpu as pltpu
from jax.experimental.pallas import tpu_sc as plsc

N_TIME = 8192
BINS = 10000
OUT_WIDTH = 2048
BATCH = 4

_TOTAL_ROWS = BATCH * N_TIME
_CHUNK = 8
_NBUF = 7


def _build_sc_call():
    info = plsc.get_sparse_core_info()
    nc, ns, nl = info.num_cores, info.num_subcores, info.num_lanes
    nw = nc * ns
    rows_per_w = _TOTAL_ROWS // nw
    n_chunks = rows_per_w // _CHUNK
    n_groups = n_chunks // _NBUF
    n_ring = n_groups * _NBUF

    mesh = plsc.VectorSubcoreMesh(core_axis_name="c", subcore_axis_name="s")

    @functools.partial(
        pl.kernel,
        mesh=mesh,
        out_type=jax.ShapeDtypeStruct((_TOTAL_ROWS, OUT_WIDTH), jnp.float32),
        scratch_types=(
            [pltpu.VMEM((2, 16), jnp.float32),
             pltpu.VMEM((rows_per_w,), jnp.int32)]
            + [pltpu.VMEM((_CHUNK, OUT_WIDTH), jnp.float32) for _ in range(_NBUF)]
            + [pltpu.SemaphoreType.DMA for _ in range(2 * _NBUF)]
        ),
    )
    def sc_kernel(params_hbm, table_hbm, out_hbm, params_v, idx_v, *rest):
        bufs = rest[:_NBUF]
        gsem = rest[_NBUF:2 * _NBUF]
        wsem = rest[2 * _NBUF:]

        wid = lax.axis_index("s") * nc + lax.axis_index("c")
        base0 = wid * rows_per_w

        pltpu.sync_copy(params_hbm.at[wid], params_v)
        sv = params_v[0, :]
        dv = params_v[1, :]

        lane = lax.iota(jnp.int32, nl).astype(jnp.float32)
        t_start = (base0 % N_TIME).astype(jnp.float32)

        def idx_body(i, _):
            tv = t_start + (i * nl).astype(jnp.float32) + lane
            pos = sv + dv * (tv * (1.0 / N_TIME))
            idx_v[pl.ds(i * nl, nl)] = (pos * float(BINS)).astype(jnp.int32)
            return 0

        lax.fori_loop(0, rows_per_w // nl, idx_body, 0)

        def gather_desc(j, b):
            return pltpu.make_async_copy(
                table_hbm.at[idx_v.at[pl.ds(j * _CHUNK, _CHUNK)]],
                bufs[b],
                gsem[b],
            )

        for b in range(_NBUF):
            gather_desc(b, b).start()

        def body(g, _):
            writes = []
            for b in range(_NBUF):
                j = g * _NBUF + b
                gather_desc(j, b).wait()
                wd = pltpu.make_async_copy(
                    bufs[b],
                    out_hbm.at[pl.ds(base0 + j * _CHUNK, _CHUNK)],
                    wsem[b],
                )
                wd.start()
                writes.append(wd)
            for b in range(_NBUF):
                jn = (g + 1) * _NBUF + b
                writes[b].wait()

                @pl.when(jn < n_ring)
                def _():
                    gather_desc(jn, b).start()

            return 0

        lax.fori_loop(0, n_groups, body, 0)

        for j in range(n_ring, n_chunks):
            gd = gather_desc(j, 0)
            gd.start()
            gd.wait()
            wd = pltpu.make_async_copy(
                bufs[0],
                out_hbm.at[pl.ds(base0 + j * _CHUNK, _CHUNK)],
                wsem[0],
            )
            wd.start()
            wd.wait()

    return sc_kernel


def kernel(pos_start, pos_end, emb_weight):
    s = pos_start.reshape(BATCH)
    d = pos_end.reshape(BATCH) - s
    s_rep = jnp.repeat(s, 8)
    d_rep = jnp.repeat(d, 8)
    params = jnp.stack([s_rep, d_rep], axis=1)
    params = jnp.broadcast_to(params[:, :, None], (32, 2, 16))
    sc_call = _build_sc_call()
    out = sc_call(params, emb_weight)
    return out.reshape(BATCH, N_TIME, OUT_WIDTH)
